# Initial kernel scaffold; baseline (speedup 1.0000x reference)
import jax
import jax.numpy as jnp
from jax.experimental import pallas as pl


def kernel(action, table):
    raise NotImplementedError("write your pallas kernel here")


# speedup vs baseline: 10.4992x; 10.4992x over previous
import functools

import jax
import jax.numpy as jnp
from jax import lax
from jax.experimental import pallas as pl
from jax.experimental.pallas import tpu as pltpu
from jax.experimental.pallas import tpu_sc as plsc

NC = 2
NS = 16
NW = NC * NS
GROUP = 4
IDX_MINOR = 128
CHUNK = 256


def _gather_kernel(n_chunks, tbl_hbm, idx_hbm, out_hbm, idx_v, rows_v, gsem):
    wid = lax.axis_index("s") * NC + lax.axis_index("c")
    k = CHUNK // IDX_MINOR
    rows_per_worker = n_chunks * CHUNK
    row_base = wid * rows_per_worker
    idx_rows = rows_per_worker // IDX_MINOR

    irow = pl.multiple_of(wid * idx_rows, 8)
    pltpu.sync_copy(idx_hbm.at[pl.ds(irow, idx_rows)], idx_v)

    def step(c, _):
        cbase = pl.multiple_of(row_base + c * CHUNK, CHUNK)
        copies = [
            pltpu.async_copy(
                tbl_hbm.at[idx_v.at[c * k + j]],
                rows_v.at[pl.ds(j * IDX_MINOR, IDX_MINOR)],
                gsem,
            )
            for j in range(k)
        ]
        for cp in copies:
            cp.wait()
        pltpu.sync_copy(rows_v, out_hbm.at[pl.ds(cbase, CHUNK)])
        return ()

    lax.fori_loop(0, n_chunks, step, (), unroll=False)


def kernel(action, table):
    B, L = action.shape
    D = table.shape[1]
    V = table.shape[0]
    R = (B * L) // GROUP
    W = D * GROUP
    assert R % (NW * CHUNK) == 0
    n_chunks = R // (NW * CHUNK)

    weights = jnp.array([V ** 3, V ** 2, V, 1], dtype=jnp.int32)
    qidx = (action.reshape(R, GROUP) * weights).sum(axis=-1)
    idx2d = qidx.reshape(R // IDX_MINOR, IDX_MINOR)

    parts = [
        jnp.broadcast_to(
            table.reshape((1,) * g + (V,) + (1,) * (GROUP - 1 - g) + (D,)),
            (V,) * GROUP + (D,),
        )
        for g in range(GROUP)
    ]
    qtab = jnp.concatenate(parts, axis=-1).reshape(V ** GROUP, W)

    mesh = plsc.VectorSubcoreMesh(core_axis_name="c", subcore_axis_name="s",
                                  num_cores=NC, num_subcores=NS)
    out = pl.kernel(
        functools.partial(_gather_kernel, n_chunks),
        out_type=jax.ShapeDtypeStruct((R, W), jnp.float32),
        mesh=mesh,
        scratch_types=[
            pltpu.VMEM((R // (NW * IDX_MINOR), IDX_MINOR), jnp.int32),
            pltpu.VMEM((CHUNK, W), jnp.float32),
            pltpu.SemaphoreType.DMA,
        ],
    )(qtab, idx2d)
    return out.reshape(B, L * D)


# speedup vs baseline: 10.5140x; 1.0014x over previous
import functools

import jax
import jax.numpy as jnp
from jax import lax
from jax.experimental import pallas as pl
from jax.experimental.pallas import tpu as pltpu
from jax.experimental.pallas import tpu_sc as plsc

NC = 2
NS = 16
NW = NC * NS
GROUP = 4
IDX_MINOR = 128
CHUNK = 256


def _gather_kernel(n_chunks, tbl_hbm, idx_hbm, out_hbm, idx_v,
                   rows_v0, rows_v1, gsem0, gsem1, ssem0, ssem1):
    wid = lax.axis_index("s") * NC + lax.axis_index("c")
    k = CHUNK // IDX_MINOR
    rows_per_worker = n_chunks * CHUNK
    row_base = wid * rows_per_worker
    idx_rows = rows_per_worker // IDX_MINOR

    irow = pl.multiple_of(wid * idx_rows, 8)
    pltpu.sync_copy(idx_hbm.at[pl.ds(irow, idx_rows)], idx_v)

    bufs = (rows_v0, rows_v1)
    gsems = (gsem0, gsem1)
    ssems = (ssem0, ssem1)

    def out_slice(c):
        cbase = pl.multiple_of(row_base + c * CHUNK, CHUNK)
        return out_hbm.at[pl.ds(cbase, CHUNK)]

    def fire_gather(c, b):
        for j in range(k):
            pltpu.async_copy(
                tbl_hbm.at[idx_v.at[c * k + j]],
                bufs[b].at[pl.ds(j * IDX_MINOR, IDX_MINOR)],
                gsems[b],
            )

    def wait_gather(b):
        for j in range(k):
            pltpu.make_async_copy(
                tbl_hbm.at[idx_v.at[j]],
                bufs[b].at[pl.ds(j * IDX_MINOR, IDX_MINOR)],
                gsems[b],
            ).wait()

    def wait_store(b):
        pltpu.make_async_copy(bufs[b], out_slice(0), ssems[b]).wait()

    fire_gather(0, 0)

    def step2(c2, _):
        for b in range(2):
            c = c2 * 2 + b
            nb = 1 - b

            @pl.when(c + 1 < n_chunks)
            def _():
                @pl.when(c >= 1)
                def _():
                    wait_store(nb)
                fire_gather(c + 1, nb)

            wait_gather(b)
            pltpu.async_copy(bufs[b], out_slice(c), ssems[b])
        return ()

    assert n_chunks % 2 == 0
    lax.fori_loop(0, n_chunks // 2, step2, (), unroll=False)
    wait_store(0)
    wait_store(1)


def kernel(action, table):
    B, L = action.shape
    D = table.shape[1]
    V = table.shape[0]
    R = (B * L) // GROUP
    W = D * GROUP
    assert R % (NW * CHUNK) == 0
    n_chunks = R // (NW * CHUNK)

    weights = jnp.array([V ** 3, V ** 2, V, 1], dtype=jnp.int32)
    qidx = (action.reshape(R, GROUP) * weights).sum(axis=-1)
    idx2d = qidx.reshape(R // IDX_MINOR, IDX_MINOR)

    parts = [
        jnp.broadcast_to(
            table.reshape((1,) * g + (V,) + (1,) * (GROUP - 1 - g) + (D,)),
            (V,) * GROUP + (D,),
        )
        for g in range(GROUP)
    ]
    qtab = jnp.concatenate(parts, axis=-1).reshape(V ** GROUP, W)

    mesh = plsc.VectorSubcoreMesh(core_axis_name="c", subcore_axis_name="s",
                                  num_cores=NC, num_subcores=NS)
    out = pl.kernel(
        functools.partial(_gather_kernel, n_chunks),
        out_type=jax.ShapeDtypeStruct((R, W), jnp.float32),
        mesh=mesh,
        scratch_types=[
            pltpu.VMEM((R // (NW * IDX_MINOR), IDX_MINOR), jnp.int32),
            pltpu.VMEM((CHUNK, W), jnp.float32),
            pltpu.VMEM((CHUNK, W), jnp.float32),
            pltpu.SemaphoreType.DMA,
            pltpu.SemaphoreType.DMA,
            pltpu.SemaphoreType.DMA,
            pltpu.SemaphoreType.DMA,
        ],
    )(qtab, idx2d)
    return out.reshape(B, L * D)


# speedup vs baseline: 28.9151x; 2.7502x over previous
import jax
import jax.numpy as jnp
from jax import lax
from jax.experimental import pallas as pl
from jax.experimental.pallas import tpu as pltpu
from jax.experimental.pallas import tpu_sc as plsc

NC = 2
NS = 16
NW = NC * NS
GROUP = 4
BLK = 8
QPB = 50
GQ = BLK * QPB
NV = 6


def _gather_kernel(n_blocks, L, tbl_hbm, act_hbm, out_hbm,
                   act_v, qidx0, qidx1, rows_v0, rows_v1,
                   gsem0, gsem1, ssem0, ssem1):
    wid = lax.axis_index("s") * NC + lax.axis_index("c")
    blk_base = wid * n_blocks

    qidxs = (qidx0, qidx1)
    bufs = (rows_v0, rows_v1)
    gsems = (gsem0, gsem1)
    ssems = (ssem0, ssem1)

    def stage_fold(c, b):
        abase = pl.multiple_of((blk_base + c) * BLK, BLK)
        pltpu.sync_copy(act_hbm.at[pl.ds(abase, BLK)], act_v)
        for i in range(GQ // 16):
            lane = lax.iota(jnp.int32, 16)
            q = i * 16 + lane
            r = q // QPB
            qm = q % QPB
            col = qm * GROUP
            v = plsc.load_gather(act_v, [r, col])
            for g in range(1, GROUP):
                v = v * NV + plsc.load_gather(act_v, [r, col + g])
            plsc.store_scatter(qidxs[b], [qm * BLK + r], v)

    def fire_gather(b):
        for j in range(QPB):
            pltpu.async_copy(
                tbl_hbm.at[qidxs[b].at[pl.ds(j * BLK, BLK)]],
                bufs[b].at[:, pl.ds(j * 128, 128)],
                gsems[b],
            )

    def wait_gather(b):
        for j in range(QPB):
            pltpu.make_async_copy(
                tbl_hbm.at[qidxs[b].at[pl.ds(0, BLK)]],
                bufs[b].at[:, pl.ds(j * 128, 128)],
                gsems[b],
            ).wait()

    def wait_store(b):
        pltpu.make_async_copy(bufs[b], out_hbm.at[blk_base], ssems[b]).wait()

    def step2(c2, _):
        for b in range(2):
            c = c2 * 2 + b
            stage_fold(c, b)

            @pl.when(c >= 2)
            def _():
                wait_store(b)

            fire_gather(b)
            wait_gather(b)
            pltpu.async_copy(bufs[b], out_hbm.at[blk_base + c], ssems[b])
        return ()

    assert n_blocks % 2 == 0
    lax.fori_loop(0, n_blocks // 2, step2, (), unroll=False)
    wait_store(0)
    wait_store(1)


def kernel(action, table):
    B, L = action.shape
    D = table.shape[1]
    V = table.shape[0]
    W = D * GROUP
    n_blocks_total = B // BLK
    assert n_blocks_total % (2 * NW) == 0
    n_blocks = n_blocks_total // NW

    parts = [
        jnp.broadcast_to(
            table.reshape((1,) * g + (V,) + (1,) * (GROUP - 1 - g) + (D,)),
            (V,) * GROUP + (D,),
        )
        for g in range(GROUP)
    ]
    qtab = jnp.concatenate(parts, axis=-1).reshape(V ** GROUP, W)

    mesh = plsc.VectorSubcoreMesh(core_axis_name="c", subcore_axis_name="s",
                                  num_cores=NC, num_subcores=NS)
    out = pl.kernel(
        lambda *refs: _gather_kernel(n_blocks, L, *refs),
        out_type=jax.ShapeDtypeStruct((n_blocks_total, BLK, L * D), jnp.float32),
        mesh=mesh,
        compiler_params=pltpu.CompilerParams(needs_layout_passes=False),
        scratch_types=[
            pltpu.VMEM((BLK, L), jnp.int32),
            pltpu.VMEM((GQ,), jnp.int32),
            pltpu.VMEM((GQ,), jnp.int32),
            pltpu.VMEM((BLK, L * D), jnp.float32),
            pltpu.VMEM((BLK, L * D), jnp.float32),
            pltpu.SemaphoreType.DMA,
            pltpu.SemaphoreType.DMA,
            pltpu.SemaphoreType.DMA,
            pltpu.SemaphoreType.DMA,
        ],
    )(qtab, action)
    return out.reshape(B, L * D)


# speedup vs baseline: 51.2406x; 1.7721x over previous
import jax
import jax.numpy as jnp
from jax import lax
from jax.experimental import pallas as pl
from jax.experimental.pallas import tpu as pltpu
from jax.experimental.pallas import tpu_sc as plsc

NC = 2
NS = 16
NW = NC * NS
GROUP = 4
BLK = 8
QPB = 50
GQ = BLK * QPB
NV = 6


def _gather_kernel(n_blocks, L, tbl_hbm, act_hbm, out_hbm,
                   act_v, qidx0, qidx1, rows_v0, rows_v1, tbl_sp,
                   gsem0, gsem1, ssem0, ssem1):
    wid = lax.axis_index("s") * NC + lax.axis_index("c")
    blk_base = wid * n_blocks

    @pl.when(lax.axis_index("s") == 0)
    def _():
        pltpu.sync_copy(tbl_hbm, tbl_sp)
    plsc.subcore_barrier()

    qidxs = (qidx0, qidx1)
    bufs = (rows_v0, rows_v1)
    gsems = (gsem0, gsem1)
    ssems = (ssem0, ssem1)

    def stage_fold(c, b):
        abase = pl.multiple_of((blk_base + c) * BLK, BLK)
        pltpu.sync_copy(act_hbm.at[pl.ds(abase, BLK)], act_v)
        for i in range(GQ // 16):
            lane = lax.iota(jnp.int32, 16)
            q = i * 16 + lane
            r = q // QPB
            qm = q % QPB
            col = qm * GROUP
            v = plsc.load_gather(act_v, [r, col])
            for g in range(1, GROUP):
                v = v * NV + plsc.load_gather(act_v, [r, col + g])
            plsc.store_scatter(qidxs[b], [qm * BLK + r], v)

    def fire_gather(b):
        for j in range(QPB):
            pltpu.async_copy(
                tbl_sp.at[qidxs[b].at[pl.ds(j * BLK, BLK)]],
                bufs[b].at[:, pl.ds(j * 128, 128)],
                gsems[b],
            )

    def wait_gather(b):
        for j in range(QPB):
            pltpu.make_async_copy(
                tbl_sp.at[qidxs[b].at[pl.ds(0, BLK)]],
                bufs[b].at[:, pl.ds(j * 128, 128)],
                gsems[b],
            ).wait()

    def wait_store(b):
        pltpu.make_async_copy(bufs[b], out_hbm.at[blk_base], ssems[b]).wait()

    def step2(c2, _):
        for b in range(2):
            c = c2 * 2 + b
            stage_fold(c, b)

            @pl.when(c >= 2)
            def _():
                wait_store(b)

            fire_gather(b)
            wait_gather(b)
            pltpu.async_copy(bufs[b], out_hbm.at[blk_base + c], ssems[b])
        return ()

    assert n_blocks % 2 == 0
    lax.fori_loop(0, n_blocks // 2, step2, (), unroll=False)
    wait_store(0)
    wait_store(1)


def kernel(action, table):
    B, L = action.shape
    D = table.shape[1]
    V = table.shape[0]
    W = D * GROUP
    n_blocks_total = B // BLK
    assert n_blocks_total % (2 * NW) == 0
    n_blocks = n_blocks_total // NW

    parts = [
        jnp.broadcast_to(
            table.reshape((1,) * g + (V,) + (1,) * (GROUP - 1 - g) + (D,)),
            (V,) * GROUP + (D,),
        )
        for g in range(GROUP)
    ]
    qtab = jnp.concatenate(parts, axis=-1).reshape(V ** GROUP, W)

    mesh = plsc.VectorSubcoreMesh(core_axis_name="c", subcore_axis_name="s",
                                  num_cores=NC, num_subcores=NS)
    out = pl.kernel(
        lambda *refs: _gather_kernel(n_blocks, L, *refs),
        out_type=jax.ShapeDtypeStruct((n_blocks_total, BLK, L * D), jnp.float32),
        mesh=mesh,
        compiler_params=pltpu.CompilerParams(needs_layout_passes=False),
        scratch_types=[
            pltpu.VMEM((BLK, L), jnp.int32),
            pltpu.VMEM((GQ,), jnp.int32),
            pltpu.VMEM((GQ,), jnp.int32),
            pltpu.VMEM((BLK, L * D), jnp.float32),
            pltpu.VMEM((BLK, L * D), jnp.float32),
            pltpu.VMEM_SHARED((V ** GROUP, W), jnp.float32),
            pltpu.SemaphoreType.DMA,
            pltpu.SemaphoreType.DMA,
            pltpu.SemaphoreType.DMA,
            pltpu.SemaphoreType.DMA,
        ],
    )(qtab, action)
    return out.reshape(B, L * D)
